# baseline (device time: 29934 ns/iter reference)
import jax
import jax.numpy as jnp
from jax import lax
from jax.experimental import pallas as pl
from jax.experimental.pallas import tpu as pltpu

N_DEV = 4
N_PEERS = N_DEV - 1
N_LAYERS = 3


def kernel(x, Win0, Wout0, Win1, Wout1, Win2, Wout2):
    b, d = x.shape

    def body(x_ref, win0, wout0, win1, wout1, win2, wout2,
             out_ref, send_buf, recv_buf, send_sems, recv_sems):
        my = lax.axis_index("i")
        wins = [win0, win1, win2]
        wouts = [wout0, wout1, wout2]

        barrier_sem = pltpu.get_barrier_semaphore()
        for idx in range(N_PEERS):
            j = (my + 1 + idx) % N_DEV
            pl.semaphore_signal(
                barrier_sem, inc=1,
                device_id=(j,), device_id_type=pl.DeviceIdType.MESH,
            )

        x_cur = x_ref[:, :]
        for L in range(N_LAYERS):
            h = jnp.maximum(
                jnp.dot(x_cur, wins[L][:, :], preferred_element_type=jnp.float32),
                0.0,
            )
            partial = jnp.dot(h, wouts[L][:, :], preferred_element_type=jnp.float32)
            send_buf[L, :, :] = partial

            if L == 0:
                pl.semaphore_wait(barrier_sem, N_PEERS)

            sends = []
            for idx in range(N_PEERS):
                j = (my + 1 + idx) % N_DEV
                rdma = pltpu.make_async_remote_copy(
                    src_ref=send_buf.at[L],
                    dst_ref=recv_buf.at[L, 2 - idx],
                    send_sem=send_sems.at[L, idx],
                    recv_sem=recv_sems.at[L, 2 - idx],
                    device_id=(j,),
                    device_id_type=pl.DeviceIdType.MESH,
                )
                rdma.start()
                sends.append(rdma)

            acc = partial
            for r in range(N_PEERS):
                recv = pltpu.make_async_remote_copy(
                    src_ref=send_buf.at[L],
                    dst_ref=recv_buf.at[L, r],
                    send_sem=send_sems.at[L, r],
                    recv_sem=recv_sems.at[L, r],
                    device_id=(my,),
                    device_id_type=pl.DeviceIdType.MESH,
                )
                recv.wait_recv()
                acc = acc + recv_buf[L, r]

            for rdma in sends:
                rdma.wait_send()

            x_cur = acc

        out_ref[:, :] = x_cur

    return pl.pallas_call(
        body,
        out_shape=jax.ShapeDtypeStruct((b, d), jnp.float32),
        in_specs=[pl.BlockSpec(memory_space=pltpu.VMEM)] * 7,
        out_specs=pl.BlockSpec(memory_space=pltpu.VMEM),
        scratch_shapes=[
            pltpu.VMEM((N_LAYERS, b, d), jnp.float32),
            pltpu.VMEM((N_LAYERS, N_PEERS, b, d), jnp.float32),
            pltpu.SemaphoreType.DMA((N_LAYERS, N_PEERS)),
            pltpu.SemaphoreType.DMA((N_LAYERS, N_PEERS)),
        ],
        compiler_params=pltpu.CompilerParams(collective_id=0),
    )(x, Win0, Wout0, Win1, Wout1, Win2, Wout2)


# device time: 29886 ns/iter; 1.0016x vs baseline; 1.0016x over previous
import jax
import jax.numpy as jnp
from jax import lax
from jax.experimental import pallas as pl
from jax.experimental.pallas import tpu as pltpu

N_DEV = 4
N_PEERS = N_DEV - 1
N_LAYERS = 3
N_CHUNKS = 4


def kernel(x, Win0, Wout0, Win1, Wout1, Win2, Wout2):
    b, d = x.shape
    cw = d // N_CHUNKS

    def body(x_ref, win0, wout0, win1, wout1, win2, wout2,
             out_ref, send_buf, recv_buf, send_sems, recv_sems):
        my = lax.axis_index("i")
        wins = [win0, win1, win2]
        wouts = [wout0, wout1, wout2]

        barrier_sem = pltpu.get_barrier_semaphore()
        for idx in range(N_PEERS):
            j = (my + 1 + idx) % N_DEV
            pl.semaphore_signal(
                barrier_sem, inc=1,
                device_id=(j,), device_id_type=pl.DeviceIdType.MESH,
            )

        h_pre = jnp.dot(x_ref[:, :], win0[:, :], preferred_element_type=jnp.float32)

        for L in range(N_LAYERS):
            h = jnp.maximum(h_pre, 0.0)

            sends = []
            for c in range(N_CHUNKS):
                cs = pl.ds(c * cw, cw)
                p_c = jnp.dot(h, wouts[L][:, cs],
                              preferred_element_type=jnp.float32)
                send_buf[L, :, cs] = p_c
                if L == 0 and c == 0:
                    pl.semaphore_wait(barrier_sem, N_PEERS)
                for idx in range(N_PEERS):
                    j = (my + 1 + idx) % N_DEV
                    rdma = pltpu.make_async_remote_copy(
                        src_ref=send_buf.at[L, :, cs],
                        dst_ref=recv_buf.at[L, 2 - idx, :, cs],
                        send_sem=send_sems.at[idx, c],
                        recv_sem=recv_sems.at[L, 2 - idx, c],
                        device_id=(j,),
                        device_id_type=pl.DeviceIdType.MESH,
                    )
                    rdma.start()
                    sends.append(rdma)

            h_pre_next = None
            for c in range(N_CHUNKS):
                cs = pl.ds(c * cw, cw)
                for r in range(N_PEERS):
                    recv = pltpu.make_async_remote_copy(
                        src_ref=send_buf.at[L, :, cs],
                        dst_ref=recv_buf.at[L, r, :, cs],
                        send_sem=send_sems.at[r, c],
                        recv_sem=recv_sems.at[L, r, c],
                        device_id=(my,),
                        device_id_type=pl.DeviceIdType.MESH,
                    )
                    recv.wait_recv()
                x_c = send_buf[L, :, cs]
                for r in range(N_PEERS):
                    x_c = x_c + recv_buf[L, r, :, cs]
                if L < N_LAYERS - 1:
                    g = jnp.dot(x_c, wins[L + 1][cs, :],
                                preferred_element_type=jnp.float32)
                    h_pre_next = g if h_pre_next is None else h_pre_next + g
                else:
                    out_ref[:, cs] = x_c

            for rdma in sends:
                rdma.wait_send()
            h_pre = h_pre_next

    return pl.pallas_call(
        body,
        out_shape=jax.ShapeDtypeStruct((b, d), jnp.float32),
        in_specs=[pl.BlockSpec(memory_space=pltpu.VMEM)] * 7,
        out_specs=pl.BlockSpec(memory_space=pltpu.VMEM),
        scratch_shapes=[
            pltpu.VMEM((N_LAYERS, b, d), jnp.float32),
            pltpu.VMEM((N_LAYERS, N_PEERS, b, d), jnp.float32),
            pltpu.SemaphoreType.DMA((N_PEERS, N_CHUNKS)),
            pltpu.SemaphoreType.DMA((N_LAYERS, N_PEERS, N_CHUNKS)),
        ],
        compiler_params=pltpu.CompilerParams(collective_id=0),
    )(x, Win0, Wout0, Win1, Wout1, Win2, Wout2)
